# Initial kernel scaffold; baseline (speedup 1.0000x reference)
import jax
import jax.numpy as jnp
from jax.experimental import pallas as pl


def kernel(x, edge_index, batch, W1, b1, W2, b2):
    raise NotImplementedError("write your pallas kernel here")


# speedup vs baseline: 16.2119x; 16.2119x over previous
import functools

import jax
import jax.numpy as jnp
from jax import lax
from jax.experimental import pallas as pl
from jax.experimental.pallas import tpu as pltpu
from jax.experimental.pallas import tpu_sc as plsc

N = 50000
E = 800000
D_IN = 79
D_HID = 64
G = 256
NC = 2
NS = 16
CH = D_HID // NC
CB = 128
NCHUNK = E // CB
DPIECE = 400
PIECE = 80
BM = 400
MB = N // BM

_f32 = jnp.float32
_i32 = jnp.int32


@functools.cache
def _mesh():
    return plsc.VectorSubcoreMesh(
        core_axis_name="c", subcore_axis_name="s", num_cores=NC, num_subcores=NS
    )


_SC_PARAMS = pltpu.CompilerParams(use_tc_tiling_on_sc=False)


def _fill_1d(ref, n, value):
    v = jnp.full((16,), value, ref.dtype)

    @pl.loop(0, n // 16)
    def _(i):
        ref[pl.ds(i * 16, 16)] = v


def _fill_2d(ref, nrows, value):
    v = jnp.full((16,), value, ref.dtype)

    @pl.loop(0, nrows)
    def _(r):
        for h in range(CH // 16):
            ref[r, pl.ds(16 * h, 16)] = v


def _split(total, t, nbig):
    base = total // NS
    start = base * t + jnp.minimum(t, nbig)
    cnt = base + (t < nbig).astype(_i32)
    return start, cnt


def _deg_body(dst2d, pdeg, deg_sh, idst, ones_v, piece_v):
    c = lax.axis_index("c")
    t = lax.axis_index("s")
    _fill_1d(piece_v, DPIECE, 0.0)
    _fill_1d(ones_v, CB, 1.0)
    p0, pcnt = _split(N // DPIECE, t, (N // DPIECE) % NS)

    @pl.loop(p0, p0 + pcnt)
    def _(p):
        pltpu.sync_copy(piece_v, deg_sh.at[pl.ds(p * DPIECE, DPIECE)])

    plsc.subcore_barrier()

    half = NCHUNK // NC
    s0, cnt = _split(half, t, half % NS)
    s0 = s0 + c * half

    @pl.loop(s0, s0 + cnt)
    def _(j):
        pltpu.sync_copy(dst2d.at[j], idst)
        pltpu.sync_copy(ones_v, deg_sh.at[idst], add=True)

    plsc.subcore_barrier()

    @pl.loop(p0, p0 + pcnt)
    def _(p):
        pltpu.sync_copy(deg_sh.at[pl.ds(p * DPIECE, DPIECE)], piece_v)
        pltpu.sync_copy(piece_v, pdeg.at[pl.ds(c * N + p * DPIECE, DPIECE)])


@functools.cache
def _deg_call():
    return pl.kernel(
        _deg_body,
        out_type=jax.ShapeDtypeStruct((NC * N,), _f32),
        mesh=_mesh(),
        compiler_params=_SC_PARAMS,
        scratch_types=[
            pltpu.VMEM_SHARED((N,), _f32),
            pltpu.VMEM((CB,), _i32),
            pltpu.VMEM((CB,), _f32),
            pltpu.VMEM((DPIECE,), _f32),
        ],
    )


def _mm_body(x_ref, w1_ref, pdeg_ref, y_ref, dinv_ref):
    pd = pdeg_ref[...]
    deg = pd[0, 0, 0, :] + pd[1, 0, 0, :] + 1.0
    dinv = lax.rsqrt(deg)
    dinv_ref[...] = dinv.reshape(1, 1, BM)
    xw = jnp.dot(x_ref[...], w1_ref[0], preferred_element_type=_f32)
    y_ref[...] = xw * dinv[:, None]


_mm_call = pl.pallas_call(
    _mm_body,
    grid=(MB, NC),
    in_specs=[
        pl.BlockSpec((BM, D_IN), lambda m, c: (m, 0)),
        pl.BlockSpec((1, D_IN, CH), lambda m, c: (c, 0, 0)),
        pl.BlockSpec((NC, 1, 1, BM), lambda m, c: (0, m, 0, 0)),
    ],
    out_specs=[
        pl.BlockSpec((BM, CH), lambda m, c: (c * MB + m, 0)),
        pl.BlockSpec((1, 1, BM), lambda m, c: (m, 0, 0)),
    ],
    out_shape=[
        jax.ShapeDtypeStruct((NC * N, CH), _f32),
        jax.ShapeDtypeStruct((MB, 1, BM), _f32),
    ],
)


def _gnn_body(
    src2d, dst2d, yflat, dinv, batch2d, b1,
    sums_out, cnt_out,
    agg_sh, sums_sh, cnt_sh,
    isrc, idst, rows, aggv, yv, dv, bv, onesv, zv, b1v, sem,
):
    c = lax.axis_index("c")
    t = lax.axis_index("s")
    coff = c * N

    _fill_2d(aggv, PIECE, 0.0)
    _fill_1d(zv, G, 0.0)
    _fill_1d(onesv, PIECE, 1.0)
    p0, pcnt = _split(N // PIECE, t, (N // PIECE) % NS)

    @pl.loop(p0, p0 + pcnt)
    def _(p):
        pltpu.sync_copy(aggv, agg_sh.at[pl.ds(p * PIECE, PIECE)])

    pltpu.sync_copy(
        aggv.at[pl.ds(0, G // NS)], sums_sh.at[pl.ds(t * (G // NS), G // NS)]
    )

    @pl.when(t == 0)
    def _():
        pltpu.sync_copy(zv, cnt_sh)

    pltpu.sync_copy(b1.at[pl.ds(c * CH, CH)], b1v)
    plsc.subcore_barrier()

    s0, cnt = _split(NCHUNK, t, NCHUNK % NS)
    offv = jnp.full((16,), coff, _i32)

    @pl.loop(s0, s0 + cnt)
    def _(j):
        pltpu.sync_copy(src2d.at[j], isrc)
        pltpu.sync_copy(dst2d.at[j], idst)
        for k in range(CB // 16):
            isrc[pl.ds(k * 16, 16)] = isrc[pl.ds(k * 16, 16)] + offv
        pltpu.async_copy(yflat.at[isrc], rows, sem).wait()
        pltpu.sync_copy(rows, agg_sh.at[idst], add=True)

    plsc.subcore_barrier()

    b1a = b1v[pl.ds(0, 16)]
    b1b = b1v[pl.ds(16, 16)]

    @pl.loop(p0, p0 + pcnt)
    def _(p):
        row0 = p * PIECE
        pltpu.sync_copy(agg_sh.at[pl.ds(row0, PIECE)], aggv)
        pltpu.sync_copy(yflat.at[pl.ds(coff + row0, PIECE)], yv)
        pltpu.sync_copy(dinv.at[pl.ds(row0, PIECE)], dv)
        pltpu.sync_copy(batch2d.at[p], bv)

        @pl.loop(0, PIECE // 16)
        def _(g):
            dvec = dv[pl.ds(g * 16, 16)]
            for l in range(16):
                r = g * 16 + l
                dsc = dvec[l]
                a0 = aggv[r, pl.ds(0, 16)]
                y0 = yv[r, pl.ds(0, 16)]
                aggv[r, pl.ds(0, 16)] = jnp.maximum(dsc * (a0 + y0) + b1a, 0.0)
                a1 = aggv[r, pl.ds(16, 16)]
                y1 = yv[r, pl.ds(16, 16)]
                aggv[r, pl.ds(16, 16)] = jnp.maximum(dsc * (a1 + y1) + b1b, 0.0)

        pltpu.sync_copy(aggv, sums_sh.at[bv], add=True)
        pltpu.sync_copy(onesv, cnt_sh.at[bv], add=True)

    plsc.subcore_barrier()

    @pl.when(t == 0)
    def _():
        for k in range(G // CB):
            pltpu.sync_copy(sums_sh.at[pl.ds(k * CB, CB)], rows)
            pltpu.sync_copy(rows, sums_out.at[c, pl.ds(k * CB, CB)])

    @pl.when(jnp.logical_and(t == 1, c == 0))
    def _():
        pltpu.sync_copy(cnt_sh, zv)
        pltpu.sync_copy(zv, cnt_out)


@functools.cache
def _gnn_call():
    return pl.kernel(
        _gnn_body,
        out_type=(
            jax.ShapeDtypeStruct((NC, G, CH), _f32),
            jax.ShapeDtypeStruct((G,), _f32),
        ),
        mesh=_mesh(),
        compiler_params=_SC_PARAMS,
        scratch_types=[
            pltpu.VMEM_SHARED((N, CH), _f32),
            pltpu.VMEM_SHARED((G, CH), _f32),
            pltpu.VMEM_SHARED((G,), _f32),
            pltpu.VMEM((CB,), _i32),
            pltpu.VMEM((CB,), _i32),
            pltpu.VMEM((CB, CH), _f32),
            pltpu.VMEM((PIECE, CH), _f32),
            pltpu.VMEM((PIECE, CH), _f32),
            pltpu.VMEM((PIECE,), _f32),
            pltpu.VMEM((PIECE,), _i32),
            pltpu.VMEM((PIECE,), _f32),
            pltpu.VMEM((G,), _f32),
            pltpu.VMEM((CH,), _f32),
            pltpu.SemaphoreType.DMA,
        ],
    )


def _fc_body(sums_ref, cnt_ref, w2_ref, b2_ref, out_ref):
    s = sums_ref[...]
    w2 = w2_ref[...]
    num = jnp.dot(s[0], w2[0:CH, :], preferred_element_type=_f32)
    num = num + jnp.dot(s[1], w2[CH:D_HID, :], preferred_element_type=_f32)
    cnt = jnp.maximum(cnt_ref[...], 1.0)
    out_ref[...] = num / cnt + b2_ref[...]


_fc_call = pl.pallas_call(
    _fc_body,
    out_shape=jax.ShapeDtypeStruct((G, 1), _f32),
)


def kernel(x, edge_index, batch, W1, b1, W2, b2):
    src2d = edge_index[0].reshape(NCHUNK, CB)
    dst2d = edge_index[1].reshape(NCHUNK, CB)
    batch2d = batch.reshape(N // PIECE, PIECE)
    pdeg = _deg_call()(dst2d)
    w1s = W1.reshape(D_IN, NC, CH).transpose(1, 0, 2)
    yflat, dinv3 = _mm_call(x, w1s, pdeg.reshape(NC, MB, 1, BM))
    dinv = dinv3.reshape(N)
    sums, counts = _gnn_call()(src2d, dst2d, yflat, dinv, batch2d, b1)
    return _fc_call(sums, counts.reshape(G, 1), W2, b2.reshape(1, 1))


# speedup vs baseline: 21.7369x; 1.3408x over previous
import functools

import jax
import jax.numpy as jnp
from jax import lax
from jax.experimental import pallas as pl
from jax.experimental.pallas import tpu as pltpu
from jax.experimental.pallas import tpu_sc as plsc

N = 50000
E = 800000
D_IN = 79
D_HID = 64
G = 256
NC = 2
NS = 16
CH = D_HID // NC
CB = 128
NCHUNK = E // CB
DPIECE = 400
PIECE = 80
BM = 400
MB = N // BM

_f32 = jnp.float32
_i32 = jnp.int32


@functools.cache
def _mesh():
    return plsc.VectorSubcoreMesh(
        core_axis_name="c", subcore_axis_name="s", num_cores=NC, num_subcores=NS
    )


_SC_PARAMS = pltpu.CompilerParams(use_tc_tiling_on_sc=False)


def _fill_1d(ref, n, value):
    v = jnp.full((16,), value, ref.dtype)

    @pl.loop(0, n // 16)
    def _(i):
        ref[pl.ds(i * 16, 16)] = v


def _fill_2d(ref, nrows, value):
    v = jnp.full((16,), value, ref.dtype)

    @pl.loop(0, nrows)
    def _(r):
        for h in range(CH // 16):
            ref[r, pl.ds(16 * h, 16)] = v


def _split(total, t, nbig):
    base = total // NS
    start = base * t + jnp.minimum(t, nbig)
    cnt = base + (t < nbig).astype(_i32)
    return start, cnt


def _split_even(total, w, nworkers, nbig):
    base = (total // nworkers) & ~1
    start = base * w + 2 * jnp.minimum(w, nbig)
    cnt = base + 2 * (w < nbig).astype(_i32)
    return start, cnt


def _deg_body(dst2d, pdeg, deg_sh, idst0, idst1, ones_v, piece_v, sem):
    c = lax.axis_index("c")
    t = lax.axis_index("s")
    _fill_1d(piece_v, DPIECE, 0.0)
    _fill_1d(ones_v, CB, 1.0)
    p0, pcnt = _split(N // DPIECE, t, (N // DPIECE) % NS)

    @pl.loop(p0, p0 + pcnt)
    def _(p):
        pltpu.sync_copy(piece_v, deg_sh.at[pl.ds(p * DPIECE, DPIECE)])

    plsc.subcore_barrier()

    w = c * NS + t
    s0, cnt = _split_even(NCHUNK, w, NC * NS, 21)
    npairs = cnt // 2
    pltpu.async_copy(dst2d.at[s0], idst0, sem)

    @pl.loop(0, npairs)
    def _(i):
        j = s0 + 2 * i
        pltpu.make_async_copy(dst2d.at[j], idst0, sem).wait()
        pltpu.async_copy(dst2d.at[j + 1], idst1, sem)
        pltpu.sync_copy(ones_v, deg_sh.at[idst0], add=True)
        pltpu.make_async_copy(dst2d.at[j + 1], idst1, sem).wait()

        @pl.when(i < npairs - 1)
        def _():
            pltpu.async_copy(dst2d.at[j + 2], idst0, sem)

        pltpu.sync_copy(ones_v, deg_sh.at[idst1], add=True)

    plsc.subcore_barrier()

    @pl.loop(p0, p0 + pcnt)
    def _(p):
        pltpu.sync_copy(deg_sh.at[pl.ds(p * DPIECE, DPIECE)], piece_v)
        pltpu.sync_copy(piece_v, pdeg.at[pl.ds(c * N + p * DPIECE, DPIECE)])


@functools.cache
def _deg_call():
    return pl.kernel(
        _deg_body,
        out_type=jax.ShapeDtypeStruct((NC * N,), _f32),
        mesh=_mesh(),
        compiler_params=_SC_PARAMS,
        scratch_types=[
            pltpu.VMEM_SHARED((N,), _f32),
            pltpu.VMEM((CB,), _i32),
            pltpu.VMEM((CB,), _i32),
            pltpu.VMEM((CB,), _f32),
            pltpu.VMEM((DPIECE,), _f32),
            pltpu.SemaphoreType.DMA,
        ],
    )


def _mm_body(x_ref, w1_ref, pdeg_ref, y_ref, dinv_ref):
    pd = pdeg_ref[...]
    deg = pd[0, 0, 0, :] + pd[1, 0, 0, :] + 1.0
    dinv = lax.rsqrt(deg)
    dinv_ref[...] = dinv.reshape(1, 1, BM)
    xw = jnp.dot(x_ref[...], w1_ref[0], preferred_element_type=_f32)
    y_ref[...] = xw * dinv[:, None]


_mm_call = pl.pallas_call(
    _mm_body,
    grid=(MB, NC),
    in_specs=[
        pl.BlockSpec((BM, D_IN), lambda m, c: (m, 0)),
        pl.BlockSpec((1, D_IN, CH), lambda m, c: (c, 0, 0)),
        pl.BlockSpec((NC, 1, 1, BM), lambda m, c: (0, m, 0, 0)),
    ],
    out_specs=[
        pl.BlockSpec((BM, CH), lambda m, c: (c * MB + m, 0)),
        pl.BlockSpec((1, 1, BM), lambda m, c: (m, 0, 0)),
    ],
    out_shape=[
        jax.ShapeDtypeStruct((NC * N, CH), _f32),
        jax.ShapeDtypeStruct((MB, 1, BM), _f32),
    ],
)


def _gnn_body(
    src2d, dst2d, yflat, dinv, batch2d, b1,
    sums_out, cnt_out,
    agg_sh, sums_sh, cnt_sh,
    isrc, idst, rows, isrc1, idst1, rows1, aggv, yv, dv, bv, onesv, zv, b1v, sem,
):
    c = lax.axis_index("c")
    t = lax.axis_index("s")
    coff = c * N

    _fill_2d(aggv, PIECE, 0.0)
    _fill_1d(zv, G, 0.0)
    _fill_1d(onesv, PIECE, 1.0)
    p0, pcnt = _split(N // PIECE, t, (N // PIECE) % NS)

    @pl.loop(p0, p0 + pcnt)
    def _(p):
        pltpu.sync_copy(aggv, agg_sh.at[pl.ds(p * PIECE, PIECE)])

    pltpu.sync_copy(
        aggv.at[pl.ds(0, G // NS)], sums_sh.at[pl.ds(t * (G // NS), G // NS)]
    )

    @pl.when(t == 0)
    def _():
        pltpu.sync_copy(zv, cnt_sh)

    pltpu.sync_copy(b1.at[pl.ds(c * CH, CH)], b1v)
    plsc.subcore_barrier()

    s0, cnt = _split_even(NCHUNK, t, NS, 5)
    npairs = cnt // 2
    offv = jnp.full((16,), coff, _i32)

    def _load_idx(j, isrc_b, idst_b):
        pltpu.sync_copy(src2d.at[j], isrc_b)
        pltpu.sync_copy(dst2d.at[j], idst_b)
        for k in range(CB // 16):
            isrc_b[pl.ds(k * 16, 16)] = isrc_b[pl.ds(k * 16, 16)] + offv

    _load_idx(s0, isrc, idst)
    pltpu.async_copy(yflat.at[isrc], rows, sem)

    @pl.loop(0, npairs)
    def _(i):
        j = s0 + 2 * i
        _load_idx(j + 1, isrc1, idst1)
        pltpu.make_async_copy(yflat.at[isrc], rows, sem).wait()
        pltpu.async_copy(yflat.at[isrc1], rows1, sem)
        pltpu.sync_copy(rows, agg_sh.at[idst], add=True)

        @pl.when(i < npairs - 1)
        def _():
            _load_idx(j + 2, isrc, idst)

        pltpu.make_async_copy(yflat.at[isrc1], rows1, sem).wait()

        @pl.when(i < npairs - 1)
        def _():
            pltpu.async_copy(yflat.at[isrc], rows, sem)

        pltpu.sync_copy(rows1, agg_sh.at[idst1], add=True)

    plsc.subcore_barrier()

    b1a = b1v[pl.ds(0, 16)]
    b1b = b1v[pl.ds(16, 16)]

    @pl.loop(p0, p0 + pcnt)
    def _(p):
        row0 = p * PIECE
        pltpu.sync_copy(agg_sh.at[pl.ds(row0, PIECE)], aggv)
        pltpu.sync_copy(yflat.at[pl.ds(coff + row0, PIECE)], yv)
        pltpu.sync_copy(dinv.at[pl.ds(row0, PIECE)], dv)
        pltpu.sync_copy(batch2d.at[p], bv)

        @pl.loop(0, PIECE // 16)
        def _(g):
            dvec = dv[pl.ds(g * 16, 16)]
            for l in range(16):
                r = g * 16 + l
                dsc = dvec[l]
                a0 = aggv[r, pl.ds(0, 16)]
                y0 = yv[r, pl.ds(0, 16)]
                aggv[r, pl.ds(0, 16)] = jnp.maximum(dsc * (a0 + y0) + b1a, 0.0)
                a1 = aggv[r, pl.ds(16, 16)]
                y1 = yv[r, pl.ds(16, 16)]
                aggv[r, pl.ds(16, 16)] = jnp.maximum(dsc * (a1 + y1) + b1b, 0.0)

        pltpu.sync_copy(aggv, sums_sh.at[bv], add=True)
        pltpu.sync_copy(onesv, cnt_sh.at[bv], add=True)

    plsc.subcore_barrier()

    @pl.when(t == 0)
    def _():
        for k in range(G // CB):
            pltpu.sync_copy(sums_sh.at[pl.ds(k * CB, CB)], rows)
            pltpu.sync_copy(rows, sums_out.at[c, pl.ds(k * CB, CB)])

    @pl.when(jnp.logical_and(t == 1, c == 0))
    def _():
        pltpu.sync_copy(cnt_sh, zv)
        pltpu.sync_copy(zv, cnt_out)


@functools.cache
def _gnn_call():
    return pl.kernel(
        _gnn_body,
        out_type=(
            jax.ShapeDtypeStruct((NC, G, CH), _f32),
            jax.ShapeDtypeStruct((G,), _f32),
        ),
        mesh=_mesh(),
        compiler_params=_SC_PARAMS,
        scratch_types=[
            pltpu.VMEM_SHARED((N, CH), _f32),
            pltpu.VMEM_SHARED((G, CH), _f32),
            pltpu.VMEM_SHARED((G,), _f32),
            pltpu.VMEM((CB,), _i32),
            pltpu.VMEM((CB,), _i32),
            pltpu.VMEM((CB, CH), _f32),
            pltpu.VMEM((CB,), _i32),
            pltpu.VMEM((CB,), _i32),
            pltpu.VMEM((CB, CH), _f32),
            pltpu.VMEM((PIECE, CH), _f32),
            pltpu.VMEM((PIECE, CH), _f32),
            pltpu.VMEM((PIECE,), _f32),
            pltpu.VMEM((PIECE,), _i32),
            pltpu.VMEM((PIECE,), _f32),
            pltpu.VMEM((G,), _f32),
            pltpu.VMEM((CH,), _f32),
            pltpu.SemaphoreType.DMA,
        ],
    )


def _fc_body(sums_ref, cnt_ref, w2_ref, b2_ref, out_ref):
    s = sums_ref[...]
    w2 = w2_ref[...]
    num = jnp.dot(s[0], w2[0:CH, :], preferred_element_type=_f32)
    num = num + jnp.dot(s[1], w2[CH:D_HID, :], preferred_element_type=_f32)
    cnt = jnp.maximum(cnt_ref[...], 1.0)
    out_ref[...] = num / cnt + b2_ref[...]


_fc_call = pl.pallas_call(
    _fc_body,
    out_shape=jax.ShapeDtypeStruct((G, 1), _f32),
)


def kernel(x, edge_index, batch, W1, b1, W2, b2):
    src2d = edge_index[0].reshape(NCHUNK, CB)
    dst2d = edge_index[1].reshape(NCHUNK, CB)
    batch2d = batch.reshape(N // PIECE, PIECE)
    pdeg = _deg_call()(dst2d)
    w1s = W1.reshape(D_IN, NC, CH).transpose(1, 0, 2)
    yflat, dinv3 = _mm_call(x, w1s, pdeg.reshape(NC, MB, 1, BM))
    dinv = dinv3.reshape(N)
    sums, counts = _gnn_call()(src2d, dst2d, yflat, dinv, batch2d, b1)
    return _fc_call(sums, counts.reshape(G, 1), W2, b2.reshape(1, 1))


# speedup vs baseline: 25.6113x; 1.1782x over previous
import functools

import jax
import jax.numpy as jnp
from jax import lax
from jax.experimental import pallas as pl
from jax.experimental.pallas import tpu as pltpu
from jax.experimental.pallas import tpu_sc as plsc

N = 50000
E = 800000
D_IN = 79
D_HID = 64
G = 256
NC = 2
NS = 16
CH = D_HID // NC
CB = 128
NCHUNK = E // CB
DPIECE = 400
PIECE = 80
BM = 400
MB = N // BM

_f32 = jnp.float32
_i32 = jnp.int32


@functools.cache
def _mesh():
    return plsc.VectorSubcoreMesh(
        core_axis_name="c", subcore_axis_name="s", num_cores=NC, num_subcores=NS
    )


_SC_PARAMS = pltpu.CompilerParams(use_tc_tiling_on_sc=False)


def _fill_1d(ref, n, value):
    v = jnp.full((16,), value, ref.dtype)

    @pl.loop(0, n // 16)
    def _(i):
        ref[pl.ds(i * 16, 16)] = v


def _fill_2d(ref, nrows, value):
    v = jnp.full((16,), value, ref.dtype)

    @pl.loop(0, nrows)
    def _(r):
        for h in range(CH // 16):
            ref[r, pl.ds(16 * h, 16)] = v


def _split(total, t, nbig):
    base = total // NS
    start = base * t + jnp.minimum(t, nbig)
    cnt = base + (t < nbig).astype(_i32)
    return start, cnt


def _split_even(total, w, nworkers, nbig):
    base = (total // nworkers) & ~1
    start = base * w + 2 * jnp.minimum(w, nbig)
    cnt = base + 2 * (w < nbig).astype(_i32)
    return start, cnt


def _deg_body(dst2d, pdeg, deg_sh, idst0, idst1, ones_v, piece_v, sem):
    c = lax.axis_index("c")
    t = lax.axis_index("s")
    _fill_1d(piece_v, DPIECE, 0.0)
    _fill_1d(ones_v, CB, 1.0)
    p0, pcnt = _split(N // DPIECE, t, (N // DPIECE) % NS)

    @pl.loop(p0, p0 + pcnt)
    def _(p):
        pltpu.sync_copy(piece_v, deg_sh.at[pl.ds(p * DPIECE, DPIECE)])

    plsc.subcore_barrier()

    w = c * NS + t
    s0, cnt = _split_even(NCHUNK, w, NC * NS, 21)
    npairs = cnt // 2
    pltpu.async_copy(dst2d.at[s0], idst0, sem)

    @pl.loop(0, npairs)
    def _(i):
        j = s0 + 2 * i
        pltpu.make_async_copy(dst2d.at[j], idst0, sem).wait()
        pltpu.async_copy(dst2d.at[j + 1], idst1, sem)
        pltpu.sync_copy(ones_v, deg_sh.at[idst0], add=True)
        pltpu.make_async_copy(dst2d.at[j + 1], idst1, sem).wait()

        @pl.when(i < npairs - 1)
        def _():
            pltpu.async_copy(dst2d.at[j + 2], idst0, sem)

        pltpu.sync_copy(ones_v, deg_sh.at[idst1], add=True)

    plsc.subcore_barrier()

    @pl.loop(p0, p0 + pcnt)
    def _(p):
        pltpu.sync_copy(deg_sh.at[pl.ds(p * DPIECE, DPIECE)], piece_v)
        pltpu.sync_copy(piece_v, pdeg.at[pl.ds(c * N + p * DPIECE, DPIECE)])


@functools.cache
def _deg_call():
    return pl.kernel(
        _deg_body,
        out_type=jax.ShapeDtypeStruct((NC * N,), _f32),
        mesh=_mesh(),
        compiler_params=_SC_PARAMS,
        scratch_types=[
            pltpu.VMEM_SHARED((N,), _f32),
            pltpu.VMEM((CB,), _i32),
            pltpu.VMEM((CB,), _i32),
            pltpu.VMEM((CB,), _f32),
            pltpu.VMEM((DPIECE,), _f32),
            pltpu.SemaphoreType.DMA,
        ],
    )


def _mm_body(x_ref, w1_ref, pdeg_ref, y_ref, dinv_ref):
    pd = pdeg_ref[...]
    deg = pd[0, 0, 0, :] + pd[1, 0, 0, :] + 1.0
    dinv = lax.rsqrt(deg)
    dinv_ref[...] = dinv.reshape(1, 1, BM)
    xw = jnp.dot(x_ref[...], w1_ref[...], preferred_element_type=_f32)
    y = xw * dinv[:, None]
    y_ref[0] = y[:, 0:CH]
    y_ref[1] = y[:, CH:D_HID]


_mm_call = pl.pallas_call(
    _mm_body,
    grid=(MB,),
    in_specs=[
        pl.BlockSpec((BM, D_IN), lambda m: (m, 0)),
        pl.BlockSpec((D_IN, D_HID), lambda m: (0, 0)),
        pl.BlockSpec((NC, 1, 1, BM), lambda m: (0, m, 0, 0)),
    ],
    out_specs=[
        pl.BlockSpec((NC, BM, CH), lambda m: (0, m, 0)),
        pl.BlockSpec((1, 1, BM), lambda m: (m, 0, 0)),
    ],
    out_shape=[
        jax.ShapeDtypeStruct((NC, N, CH), _f32),
        jax.ShapeDtypeStruct((MB, 1, BM), _f32),
    ],
)


def _gnn_body(
    edges, yflat, dinv, batch, b1,
    sums_out, cnt_out,
    agg_sh, sums_sh, cnt_sh,
    isrc0, idst0, rows0, isrc1, idst1, rows1,
    isrc2, idst2, rows2, isrc3, idst3, rows3,
    aggv, yv, dv, bv, onesv, zv, b1v, sem, sem_s,
):
    c = lax.axis_index("c")
    t = lax.axis_index("s")
    coff = c * N

    _fill_2d(aggv, PIECE, 0.0)
    _fill_1d(zv, G, 0.0)
    _fill_1d(onesv, PIECE, 1.0)
    p0, pcnt = _split(N // PIECE, t, (N // PIECE) % NS)

    @pl.loop(p0, p0 + pcnt)
    def _(p):
        pltpu.sync_copy(aggv, agg_sh.at[pl.ds(p * PIECE, PIECE)])

    pltpu.sync_copy(
        aggv.at[pl.ds(0, G // NS)], sums_sh.at[pl.ds(t * (G // NS), G // NS)]
    )

    @pl.when(t == 0)
    def _():
        pltpu.sync_copy(zv, cnt_sh)

    pltpu.sync_copy(b1.at[pl.ds(c * CH, CH)], b1v)
    plsc.subcore_barrier()

    nq = (NCHUNK - 2) // 4
    q0, qcnt = _split(nq, t, nq % NS)
    s0 = 4 * q0
    offv = jnp.full((16,), coff, _i32)
    slots = (
        (isrc0, idst0, rows0),
        (isrc1, idst1, rows1),
        (isrc2, idst2, rows2),
        (isrc3, idst3, rows3),
    )

    def _load_idx(j, si, di):
        pltpu.sync_copy(edges.at[0, pl.ds(j * CB, CB)], si)
        pltpu.sync_copy(edges.at[1, pl.ds(j * CB, CB)], di)
        for k in range(CB // 16):
            si[pl.ds(k * 16, 16)] = si[pl.ds(k * 16, 16)] + offv

    for k in range(3):
        si, di, rv = slots[k]
        _load_idx(s0 + k, si, di)
        pltpu.async_copy(yflat.at[si], rv, sem)

    @pl.loop(0, qcnt)
    def _(i):
        for k in range(4):
            si, di, rv = slots[k]
            sn, dn, rn = slots[(k + 3) % 4]
            m = s0 + 4 * i + k
            pltpu.make_async_copy(yflat.at[si], rv, sem).wait()
            pltpu.make_async_copy(rv, agg_sh.at[di], sem_s).start(add=True)
            if k == 0:
                @pl.when(i > 0)
                def _():
                    pltpu.make_async_copy(rn, agg_sh.at[dn], sem_s).wait()

                _load_idx(m + 3, sn, dn)
                pltpu.async_copy(yflat.at[sn], rn, sem)
            else:
                pltpu.make_async_copy(rn, agg_sh.at[dn], sem_s).wait()

                @pl.when(i < qcnt - 1)
                def _():
                    _load_idx(m + 3, sn, dn)
                    pltpu.async_copy(yflat.at[sn], rn, sem)

    pltpu.make_async_copy(rows3, agg_sh.at[idst3], sem_s).wait()

    @pl.when(t == NS - 1)
    def _():
        for j in (4 * nq, 4 * nq + 1):
            _load_idx(j, isrc0, idst0)
            pltpu.async_copy(yflat.at[isrc0], rows0, sem).wait()
            pltpu.sync_copy(rows0, agg_sh.at[idst0], add=True)

    plsc.subcore_barrier()

    b1a = b1v[pl.ds(0, 16)]
    b1b = b1v[pl.ds(16, 16)]

    @pl.loop(p0, p0 + pcnt)
    def _(p):
        row0 = p * PIECE
        pltpu.sync_copy(agg_sh.at[pl.ds(row0, PIECE)], aggv)
        pltpu.sync_copy(yflat.at[pl.ds(coff + row0, PIECE)], yv)
        pltpu.sync_copy(dinv.at[pl.ds(row0, PIECE)], dv)
        pltpu.sync_copy(batch.at[pl.ds(row0, PIECE)], bv)

        @pl.loop(0, PIECE // 16)
        def _(g):
            dvec = dv[pl.ds(g * 16, 16)]
            for l in range(16):
                r = g * 16 + l
                dsc = dvec[l]
                a0 = aggv[r, pl.ds(0, 16)]
                y0 = yv[r, pl.ds(0, 16)]
                aggv[r, pl.ds(0, 16)] = jnp.maximum(dsc * (a0 + y0) + b1a, 0.0)
                a1 = aggv[r, pl.ds(16, 16)]
                y1 = yv[r, pl.ds(16, 16)]
                aggv[r, pl.ds(16, 16)] = jnp.maximum(dsc * (a1 + y1) + b1b, 0.0)

        pltpu.sync_copy(aggv, sums_sh.at[bv], add=True)
        pltpu.sync_copy(onesv, cnt_sh.at[bv], add=True)

    plsc.subcore_barrier()

    @pl.when(t == 0)
    def _():
        for k in range(G // CB):
            pltpu.sync_copy(sums_sh.at[pl.ds(k * CB, CB)], rows0)
            pltpu.sync_copy(rows0, sums_out.at[c, pl.ds(k * CB, CB)])

    @pl.when(jnp.logical_and(t == 1, c == 0))
    def _():
        pltpu.sync_copy(cnt_sh, zv)
        pltpu.sync_copy(zv, cnt_out)


@functools.cache
def _gnn_call():
    return pl.kernel(
        _gnn_body,
        out_type=(
            jax.ShapeDtypeStruct((NC, G, CH), _f32),
            jax.ShapeDtypeStruct((G,), _f32),
        ),
        mesh=_mesh(),
        compiler_params=_SC_PARAMS,
        scratch_types=[
            pltpu.VMEM_SHARED((N, CH), _f32),
            pltpu.VMEM_SHARED((G, CH), _f32),
            pltpu.VMEM_SHARED((G,), _f32),
            pltpu.VMEM((CB,), _i32),
            pltpu.VMEM((CB,), _i32),
            pltpu.VMEM((CB, CH), _f32),
            pltpu.VMEM((CB,), _i32),
            pltpu.VMEM((CB,), _i32),
            pltpu.VMEM((CB, CH), _f32),
            pltpu.VMEM((CB,), _i32),
            pltpu.VMEM((CB,), _i32),
            pltpu.VMEM((CB, CH), _f32),
            pltpu.VMEM((CB,), _i32),
            pltpu.VMEM((CB,), _i32),
            pltpu.VMEM((CB, CH), _f32),
            pltpu.VMEM((PIECE, CH), _f32),
            pltpu.VMEM((PIECE, CH), _f32),
            pltpu.VMEM((PIECE,), _f32),
            pltpu.VMEM((PIECE,), _i32),
            pltpu.VMEM((PIECE,), _f32),
            pltpu.VMEM((G,), _f32),
            pltpu.VMEM((CH,), _f32),
            pltpu.SemaphoreType.DMA,
            pltpu.SemaphoreType.DMA,
        ],
    )


def _fc_body(sums_ref, cnt_ref, w2_ref, b2_ref, out_ref):
    s = sums_ref[...]
    w2 = w2_ref[...]
    num = jnp.dot(s[0], w2[0:CH, :], preferred_element_type=_f32)
    num = num + jnp.dot(s[1], w2[CH:D_HID, :], preferred_element_type=_f32)
    cnt = jnp.maximum(cnt_ref[...], 1.0)
    out_ref[...] = num / cnt + b2_ref[...]


_fc_call = pl.pallas_call(
    _fc_body,
    out_shape=jax.ShapeDtypeStruct((G, 1), _f32),
)


def kernel(x, edge_index, batch, W1, b1, W2, b2):
    dst2d = edge_index[1].reshape(NCHUNK, CB)
    pdeg = _deg_call()(dst2d)
    y3, dinv3 = _mm_call(x, W1, pdeg.reshape(NC, MB, 1, BM))
    yflat = y3.reshape(NC * N, CH)
    dinv = dinv3.reshape(N)
    sums, counts = _gnn_call()(edge_index, yflat, dinv, batch, b1)
    return _fc_call(sums, counts.reshape(G, 1), W2, b2.reshape(1, 1))


# speedup vs baseline: 36.5529x; 1.4272x over previous
import functools

import jax
import jax.numpy as jnp
from jax import lax
from jax.experimental import pallas as pl
from jax.experimental.pallas import tpu as pltpu
from jax.experimental.pallas import tpu_sc as plsc

N = 50000
E = 800000
D_IN = 79
D_HID = 64
G = 256
NC = 2
NS = 16
CH = D_HID // NC
CB = 128
NCHUNK = E // CB
KF = 2
FB = KF * CB
NFAT = E // FB
DPIECE = 400
PIECE = 80
BM = 400
MB = N // BM

_f32 = jnp.float32
_i32 = jnp.int32


@functools.cache
def _mesh():
    return plsc.VectorSubcoreMesh(
        core_axis_name="c", subcore_axis_name="s", num_cores=NC, num_subcores=NS
    )


_SC_PARAMS = pltpu.CompilerParams(use_tc_tiling_on_sc=False)


def _fill_1d(ref, n, value):
    v = jnp.full((16,), value, ref.dtype)

    @pl.loop(0, n // 16)
    def _(i):
        ref[pl.ds(i * 16, 16)] = v


def _fill_2d(ref, nrows, value):
    v = jnp.full((16,), value, ref.dtype)

    @pl.loop(0, nrows)
    def _(r):
        for h in range(CH // 16):
            ref[r, pl.ds(16 * h, 16)] = v


def _split(total, t, nbig):
    base = total // NS
    start = base * t + jnp.minimum(t, nbig)
    cnt = base + (t < nbig).astype(_i32)
    return start, cnt


def _splitw(total, w, n):
    base = total // n
    nbig = total % n
    start = base * w + jnp.minimum(w, nbig)
    cnt = base + (w < nbig).astype(_i32)
    return start, cnt


def _deg_body(dst1d, pdeg, deg_sh, idst0, idst1, ones_v, piece_v, sem):
    c = lax.axis_index("c")
    t = lax.axis_index("s")
    _fill_1d(piece_v, DPIECE, 0.0)
    _fill_1d(ones_v, FB, 1.0)
    p0, pcnt = _split(N // DPIECE, t, (N // DPIECE) % NS)

    @pl.loop(p0, p0 + pcnt)
    def _(p):
        pltpu.sync_copy(piece_v, deg_sh.at[pl.ds(p * DPIECE, DPIECE)])

    plsc.subcore_barrier()

    w = c * NS + t
    f0, cnt = _splitw(NFAT, w, NC * NS)
    npairs = cnt // 2

    def _idx(j):
        return dst1d.at[pl.ds(j * FB, FB)]

    pltpu.async_copy(_idx(f0), idst0, sem)

    @pl.loop(0, npairs)
    def _(i):
        j = f0 + 2 * i
        pltpu.make_async_copy(_idx(j), idst0, sem).wait()
        pltpu.async_copy(_idx(j + 1), idst1, sem)
        pltpu.sync_copy(ones_v, deg_sh.at[idst0], add=True)
        pltpu.make_async_copy(_idx(j + 1), idst1, sem).wait()

        @pl.when(2 * i + 2 < cnt)
        def _():
            pltpu.async_copy(_idx(j + 2), idst0, sem)

        pltpu.sync_copy(ones_v, deg_sh.at[idst1], add=True)

    @pl.when(cnt % 2 == 1)
    def _():
        jl = f0 + cnt - 1
        pltpu.make_async_copy(_idx(jl), idst0, sem).wait()
        pltpu.sync_copy(ones_v, deg_sh.at[idst0], add=True)

    plsc.subcore_barrier()

    @pl.loop(p0, p0 + pcnt)
    def _(p):
        pltpu.sync_copy(deg_sh.at[pl.ds(p * DPIECE, DPIECE)], piece_v)
        pltpu.sync_copy(piece_v, pdeg.at[pl.ds(c * N + p * DPIECE, DPIECE)])


@functools.cache
def _deg_call():
    return pl.kernel(
        _deg_body,
        out_type=jax.ShapeDtypeStruct((NC * N,), _f32),
        mesh=_mesh(),
        compiler_params=_SC_PARAMS,
        scratch_types=[
            pltpu.VMEM_SHARED((N,), _f32),
            pltpu.VMEM((FB,), _i32),
            pltpu.VMEM((FB,), _i32),
            pltpu.VMEM((FB,), _f32),
            pltpu.VMEM((DPIECE,), _f32),
            pltpu.SemaphoreType.DMA,
        ],
    )


def _mm_body(x_ref, w1_ref, pdeg_ref, y_ref, dinv_ref):
    pd = pdeg_ref[...]
    deg = pd[0, 0, 0, :] + pd[1, 0, 0, :] + 1.0
    dinv = lax.rsqrt(deg)
    dinv_ref[...] = dinv.reshape(1, 1, BM)
    xw = jnp.dot(x_ref[...], w1_ref[...], preferred_element_type=_f32)
    y = xw * dinv[:, None]
    y_ref[0] = y[:, 0:CH]
    y_ref[1] = y[:, CH:D_HID]


_mm_call = pl.pallas_call(
    _mm_body,
    grid=(MB,),
    in_specs=[
        pl.BlockSpec((BM, D_IN), lambda m: (m, 0)),
        pl.BlockSpec((D_IN, D_HID), lambda m: (0, 0)),
        pl.BlockSpec((NC, 1, 1, BM), lambda m: (0, m, 0, 0)),
    ],
    out_specs=[
        pl.BlockSpec((NC, BM, CH), lambda m: (0, m, 0)),
        pl.BlockSpec((1, 1, BM), lambda m: (m, 0, 0)),
    ],
    out_shape=[
        jax.ShapeDtypeStruct((NC, N, CH), _f32),
        jax.ShapeDtypeStruct((MB, 1, BM), _f32),
    ],
)


def _gnn_body(
    edges4, yflat, dinv, batch, b1,
    sums_out, cnt_out,
    agg_sh, sums_sh, cnt_sh,
    isrc0, isrc1, isrc2, isrc3, idst0, idst1, idst2, idst3, rows_b,
    aggv, yv, dv, bv, onesv, zv, b1v, sem, sem_s, sem_i,
):
    c = lax.axis_index("c")
    t = lax.axis_index("s")
    coff = c * N

    _fill_2d(aggv, PIECE, 0.0)
    _fill_1d(zv, G, 0.0)
    _fill_1d(onesv, PIECE, 1.0)
    p0, pcnt = _split(N // PIECE, t, (N // PIECE) % NS)

    @pl.loop(p0, p0 + pcnt)
    def _(p):
        pltpu.sync_copy(aggv, agg_sh.at[pl.ds(p * PIECE, PIECE)])

    pltpu.sync_copy(
        aggv.at[pl.ds(0, G // NS)], sums_sh.at[pl.ds(t * (G // NS), G // NS)]
    )

    @pl.when(t == 0)
    def _():
        pltpu.sync_copy(zv, cnt_sh)

    pltpu.sync_copy(b1.at[pl.ds(c * CH, CH)], b1v)
    plsc.subcore_barrier()

    nfq = (E // FB - 1) // 4
    q0, qcnt = _split(nfq, t, nfq % NS)
    f0 = 4 * q0
    offv = jnp.full((16,), coff, _i32)
    isrcs = (isrc0, isrc1, isrc2, isrc3)
    idsts = (idst0, idst1, idst2, idst3)

    def _fire_idx(f, r):
        pltpu.async_copy(edges4.at[0, pl.ds(f * FB, FB)], isrcs[r], sem_i)
        pltpu.async_copy(edges4.at[1, pl.ds(f * FB, FB)], idsts[r], sem_i)

    def _wait_idx_adjust(f, r):
        si = isrcs[r]
        pltpu.make_async_copy(edges4.at[0, pl.ds(f * FB, FB)], si, sem_i).wait()
        pltpu.make_async_copy(
            edges4.at[1, pl.ds(f * FB, FB)], idsts[r], sem_i
        ).wait()
        for k in range(FB // 16):
            si[pl.ds(k * 16, 16)] = si[pl.ds(k * 16, 16)] + offv

    def _wait_g(r, p):
        pltpu.make_async_copy(yflat.at[isrcs[r]], rows_b.at[p], sem).wait()

    def _fire_g(r, p):
        pltpu.async_copy(yflat.at[isrcs[r]], rows_b.at[p], sem)

    def _fire_s(r, p):
        pltpu.make_async_copy(rows_b.at[p], agg_sh.at[idsts[r]], sem_s).start(
            add=True
        )

    def _wait_s(r, p):
        pltpu.make_async_copy(rows_b.at[p], agg_sh.at[idsts[r]], sem_s).wait()

    for k in range(3):
        _fire_idx(f0 + k, k)
    _wait_idx_adjust(f0, 0)
    _fire_g(0, 0)

    @pl.loop(0, qcnt)
    def _(i):
        for k in range(4):
            p = k % 2
            _wait_g(k, p)
            _fire_s(k, p)
            if k == 0:
                @pl.when(i > 0)
                def _():
                    _wait_s(3, 1)
            else:
                _wait_s(k - 1, 1 - p)
            if k == 3:
                @pl.when(i < qcnt - 1)
                def _():
                    _wait_idx_adjust(f0 + 4 * i + k + 1, 0)
                    _fire_g(0, 1 - p)
            else:
                _wait_idx_adjust(f0 + 4 * i + k + 1, (k + 1) % 4)
                _fire_g((k + 1) % 4, 1 - p)
            if k == 0:
                _fire_idx(f0 + 4 * i + k + 3, 3)
            else:
                @pl.when(i < qcnt - 1)
                def _():
                    _fire_idx(f0 + 4 * i + k + 3, (k + 3) % 4)

    _wait_s(3, 1)

    @pl.when(t == NS - 1)
    def _():
        f = 4 * nfq
        _fire_idx(f, 0)
        _wait_idx_adjust(f, 0)
        _fire_g(0, 0)
        _wait_g(0, 0)
        pltpu.sync_copy(rows_b.at[0], agg_sh.at[idsts[0]], add=True)

    plsc.subcore_barrier()

    b1a = b1v[pl.ds(0, 16)]
    b1b = b1v[pl.ds(16, 16)]

    @pl.loop(p0, p0 + pcnt)
    def _(p):
        row0 = p * PIECE
        pltpu.sync_copy(agg_sh.at[pl.ds(row0, PIECE)], aggv)
        pltpu.sync_copy(yflat.at[pl.ds(coff + row0, PIECE)], yv)
        pltpu.sync_copy(dinv.at[pl.ds(row0, PIECE)], dv)
        pltpu.sync_copy(batch.at[pl.ds(row0, PIECE)], bv)

        @pl.loop(0, PIECE // 16)
        def _(g):
            dvec = dv[pl.ds(g * 16, 16)]
            for l in range(16):
                r = g * 16 + l
                dsc = dvec[l]
                a0 = aggv[r, pl.ds(0, 16)]
                y0 = yv[r, pl.ds(0, 16)]
                aggv[r, pl.ds(0, 16)] = jnp.maximum(dsc * (a0 + y0) + b1a, 0.0)
                a1 = aggv[r, pl.ds(16, 16)]
                y1 = yv[r, pl.ds(16, 16)]
                aggv[r, pl.ds(16, 16)] = jnp.maximum(dsc * (a1 + y1) + b1b, 0.0)

        pltpu.sync_copy(aggv, sums_sh.at[bv], add=True)
        pltpu.sync_copy(onesv, cnt_sh.at[bv], add=True)

    plsc.subcore_barrier()

    @pl.when(t == 0)
    def _():
        pltpu.sync_copy(sums_sh, rows_b.at[0])
        pltpu.sync_copy(rows_b.at[0], sums_out.at[c])

    @pl.when(jnp.logical_and(t == 1, c == 0))
    def _():
        pltpu.sync_copy(cnt_sh, zv)
        pltpu.sync_copy(zv, cnt_out)


@functools.cache
def _gnn_call():
    return pl.kernel(
        _gnn_body,
        out_type=(
            jax.ShapeDtypeStruct((NC, G, CH), _f32),
            jax.ShapeDtypeStruct((G,), _f32),
        ),
        mesh=_mesh(),
        compiler_params=_SC_PARAMS,
        scratch_types=[
            pltpu.VMEM_SHARED((N, CH), _f32),
            pltpu.VMEM_SHARED((G, CH), _f32),
            pltpu.VMEM_SHARED((G,), _f32),
            pltpu.VMEM((FB,), _i32),
            pltpu.VMEM((FB,), _i32),
            pltpu.VMEM((FB,), _i32),
            pltpu.VMEM((FB,), _i32),
            pltpu.VMEM((FB,), _i32),
            pltpu.VMEM((FB,), _i32),
            pltpu.VMEM((FB,), _i32),
            pltpu.VMEM((FB,), _i32),
            pltpu.VMEM((2, FB, CH), _f32),
            pltpu.VMEM((PIECE, CH), _f32),
            pltpu.VMEM((PIECE, CH), _f32),
            pltpu.VMEM((PIECE,), _f32),
            pltpu.VMEM((PIECE,), _i32),
            pltpu.VMEM((PIECE,), _f32),
            pltpu.VMEM((G,), _f32),
            pltpu.VMEM((CH,), _f32),
            pltpu.SemaphoreType.DMA,
            pltpu.SemaphoreType.DMA,
            pltpu.SemaphoreType.DMA,
        ],
    )


def _fc_body(sums_ref, cnt_ref, w2_ref, b2_ref, out_ref):
    s = sums_ref[...]
    w2 = w2_ref[...]
    num = jnp.dot(s[0], w2[0:CH, :], preferred_element_type=_f32)
    num = num + jnp.dot(s[1], w2[CH:D_HID, :], preferred_element_type=_f32)
    cnt = jnp.maximum(cnt_ref[...], 1.0)
    out_ref[...] = num / cnt + b2_ref[...]


_fc_call = pl.pallas_call(
    _fc_body,
    out_shape=jax.ShapeDtypeStruct((G, 1), _f32),
)


def kernel(x, edge_index, batch, W1, b1, W2, b2):
    pdeg = _deg_call()(edge_index[1])
    y3, dinv3 = _mm_call(x, W1, pdeg.reshape(NC, MB, 1, BM))
    yflat = y3.reshape(NC * N, CH)
    dinv = dinv3.reshape(N)
    sums, counts = _gnn_call()(edge_index, yflat, dinv, batch, b1)
    return _fc_call(sums, counts.reshape(G, 1), W2, b2.reshape(1, 1))


# speedup vs baseline: 38.7011x; 1.0588x over previous
import functools

import jax
import jax.numpy as jnp
from jax import lax
from jax.experimental import pallas as pl
from jax.experimental.pallas import tpu as pltpu
from jax.experimental.pallas import tpu_sc as plsc

N = 50000
E = 800000
D_IN = 79
D_HID = 64
G = 256
NC = 2
NS = 16
CH = D_HID // NC
CB = 128
NCHUNK = E // CB
FB = 320
NFAT = E // FB
DPIECE = 400
PIECE = 80
BM = 400
MB = N // BM

_f32 = jnp.float32
_i32 = jnp.int32


@functools.cache
def _mesh():
    return plsc.VectorSubcoreMesh(
        core_axis_name="c", subcore_axis_name="s", num_cores=NC, num_subcores=NS
    )


_SC_PARAMS = pltpu.CompilerParams(use_tc_tiling_on_sc=False)


def _fill_1d(ref, n, value):
    v = jnp.full((16,), value, ref.dtype)

    @pl.loop(0, n // 16)
    def _(i):
        ref[pl.ds(i * 16, 16)] = v


def _fill_2d(ref, nrows, value):
    v = jnp.full((16,), value, ref.dtype)

    @pl.loop(0, nrows)
    def _(r):
        for h in range(CH // 16):
            ref[r, pl.ds(16 * h, 16)] = v


def _split(total, t, nbig):
    base = total // NS
    start = base * t + jnp.minimum(t, nbig)
    cnt = base + (t < nbig).astype(_i32)
    return start, cnt


def _splitw(total, w, n):
    base = total // n
    nbig = total % n
    start = base * w + jnp.minimum(w, nbig)
    cnt = base + (w < nbig).astype(_i32)
    return start, cnt


def _deg_body(dst1d, pdeg, deg_sh, idst0, idst1, ones_v, piece_v, sem):
    c = lax.axis_index("c")
    t = lax.axis_index("s")
    _fill_1d(piece_v, DPIECE, 0.0)
    _fill_1d(ones_v, FB, 1.0)
    p0, pcnt = _split(N // DPIECE, t, (N // DPIECE) % NS)

    @pl.loop(p0, p0 + pcnt)
    def _(p):
        pltpu.sync_copy(piece_v, deg_sh.at[pl.ds(p * DPIECE, DPIECE)])

    plsc.subcore_barrier()

    w = c * NS + t
    f0, cnt = _splitw(NFAT, w, NC * NS)
    npairs = cnt // 2

    def _idx(j):
        return dst1d.at[pl.ds(j * FB, FB)]

    pltpu.async_copy(_idx(f0), idst0, sem)

    @pl.loop(0, npairs)
    def _(i):
        j = f0 + 2 * i
        pltpu.make_async_copy(_idx(j), idst0, sem).wait()
        pltpu.async_copy(_idx(j + 1), idst1, sem)
        pltpu.sync_copy(ones_v, deg_sh.at[idst0], add=True)
        pltpu.make_async_copy(_idx(j + 1), idst1, sem).wait()

        @pl.when(2 * i + 2 < cnt)
        def _():
            pltpu.async_copy(_idx(j + 2), idst0, sem)

        pltpu.sync_copy(ones_v, deg_sh.at[idst1], add=True)

    @pl.when(cnt % 2 == 1)
    def _():
        jl = f0 + cnt - 1
        pltpu.make_async_copy(_idx(jl), idst0, sem).wait()
        pltpu.sync_copy(ones_v, deg_sh.at[idst0], add=True)

    plsc.subcore_barrier()

    @pl.loop(p0, p0 + pcnt)
    def _(p):
        pltpu.sync_copy(deg_sh.at[pl.ds(p * DPIECE, DPIECE)], piece_v)
        pltpu.sync_copy(piece_v, pdeg.at[pl.ds(c * N + p * DPIECE, DPIECE)])


@functools.cache
def _deg_call():
    return pl.kernel(
        _deg_body,
        out_type=jax.ShapeDtypeStruct((NC * N,), _f32),
        mesh=_mesh(),
        compiler_params=_SC_PARAMS,
        scratch_types=[
            pltpu.VMEM_SHARED((N,), _f32),
            pltpu.VMEM((FB,), _i32),
            pltpu.VMEM((FB,), _i32),
            pltpu.VMEM((FB,), _f32),
            pltpu.VMEM((DPIECE,), _f32),
            pltpu.SemaphoreType.DMA,
        ],
    )


def _mm_body(x_ref, w1_ref, pdeg_ref, y_ref, dinv_ref):
    pd = pdeg_ref[...]
    deg = pd[0, 0, 0, :] + pd[1, 0, 0, :] + 1.0
    dinv = lax.rsqrt(deg)
    dinv_ref[...] = dinv.reshape(1, 1, BM)
    xw = jnp.dot(x_ref[...], w1_ref[...], preferred_element_type=_f32)
    y = xw * dinv[:, None]
    y_ref[0] = y[:, 0:CH]
    y_ref[1] = y[:, CH:D_HID]


_mm_call = pl.pallas_call(
    _mm_body,
    grid=(MB,),
    in_specs=[
        pl.BlockSpec((BM, D_IN), lambda m: (m, 0)),
        pl.BlockSpec((D_IN, D_HID), lambda m: (0, 0)),
        pl.BlockSpec((NC, 1, 1, BM), lambda m: (0, m, 0, 0)),
    ],
    out_specs=[
        pl.BlockSpec((NC, BM, CH), lambda m: (0, m, 0)),
        pl.BlockSpec((1, 1, BM), lambda m: (m, 0, 0)),
    ],
    out_shape=[
        jax.ShapeDtypeStruct((NC, N, CH), _f32),
        jax.ShapeDtypeStruct((MB, 1, BM), _f32),
    ],
)


def _gnn_body(
    edges4, yflat, dinv, batch, b1,
    sums_out, cnt_out,
    agg_sh, sums_sh, cnt_sh,
    isrc0, isrc1, isrc2, isrc3, idst0, idst1, idst2, idst3, rows_b,
    aggv, yv, dv, bv, onesv, zv, b1v, sem, sem_s, sem_i,
):
    c = lax.axis_index("c")
    t = lax.axis_index("s")
    coff = c * N

    _fill_2d(aggv, PIECE, 0.0)
    _fill_1d(zv, G, 0.0)
    _fill_1d(onesv, PIECE, 1.0)
    p0, pcnt = _split(N // PIECE, t, (N // PIECE) % NS)

    @pl.loop(p0, p0 + pcnt)
    def _(p):
        pltpu.sync_copy(aggv, agg_sh.at[pl.ds(p * PIECE, PIECE)])

    pltpu.sync_copy(
        aggv.at[pl.ds(0, G // NS)], sums_sh.at[pl.ds(t * (G // NS), G // NS)]
    )

    @pl.when(t == 0)
    def _():
        pltpu.sync_copy(zv, cnt_sh)

    pltpu.sync_copy(b1.at[pl.ds(c * CH, CH)], b1v)
    plsc.subcore_barrier()

    nfq = NFAT // 4
    q0, qcnt = _split(nfq, t, nfq % NS)
    f0 = 4 * q0
    offv = jnp.full((16,), coff, _i32)
    isrcs = (isrc0, isrc1, isrc2, isrc3)
    idsts = (idst0, idst1, idst2, idst3)

    def _fire_idx(f, r):
        pltpu.async_copy(edges4.at[0, pl.ds(f * FB, FB)], isrcs[r], sem_i)
        pltpu.async_copy(edges4.at[1, pl.ds(f * FB, FB)], idsts[r], sem_i)

    def _wait_idx_adjust(f, r):
        si = isrcs[r]
        pltpu.make_async_copy(edges4.at[0, pl.ds(f * FB, FB)], si, sem_i).wait()
        pltpu.make_async_copy(
            edges4.at[1, pl.ds(f * FB, FB)], idsts[r], sem_i
        ).wait()
        for k in range(FB // 16):
            si[pl.ds(k * 16, 16)] = si[pl.ds(k * 16, 16)] + offv

    def _wait_g(r, p):
        pltpu.make_async_copy(yflat.at[isrcs[r]], rows_b.at[p], sem).wait()

    def _fire_g(r, p):
        pltpu.async_copy(yflat.at[isrcs[r]], rows_b.at[p], sem)

    def _fire_s(r, p):
        pltpu.make_async_copy(rows_b.at[p], agg_sh.at[idsts[r]], sem_s).start(
            add=True
        )

    def _wait_s(r, p):
        pltpu.make_async_copy(rows_b.at[p], agg_sh.at[idsts[r]], sem_s).wait()

    for k in range(3):
        _fire_idx(f0 + k, k)
    _wait_idx_adjust(f0, 0)
    _fire_g(0, 0)

    @pl.loop(0, qcnt)
    def _(i):
        for k in range(4):
            p = k % 2
            _wait_g(k, p)
            _fire_s(k, p)
            if k == 0:
                @pl.when(i > 0)
                def _():
                    _wait_s(3, 1)
            else:
                _wait_s(k - 1, 1 - p)
            if k == 3:
                @pl.when(i < qcnt - 1)
                def _():
                    _wait_idx_adjust(f0 + 4 * i + k + 1, 0)
                    _fire_g(0, 1 - p)
            else:
                _wait_idx_adjust(f0 + 4 * i + k + 1, (k + 1) % 4)
                _fire_g((k + 1) % 4, 1 - p)
            if k == 0:
                _fire_idx(f0 + 4 * i + k + 3, 3)
            else:
                @pl.when(i < qcnt - 1)
                def _():
                    _fire_idx(f0 + 4 * i + k + 3, (k + 3) % 4)

    _wait_s(3, 1)

    plsc.subcore_barrier()

    b1a = b1v[pl.ds(0, 16)]
    b1b = b1v[pl.ds(16, 16)]

    @pl.loop(p0, p0 + pcnt)
    def _(p):
        row0 = p * PIECE
        pltpu.sync_copy(agg_sh.at[pl.ds(row0, PIECE)], aggv)
        pltpu.sync_copy(yflat.at[pl.ds(coff + row0, PIECE)], yv)
        pltpu.sync_copy(dinv.at[pl.ds(row0, PIECE)], dv)
        pltpu.sync_copy(batch.at[pl.ds(row0, PIECE)], bv)

        @pl.loop(0, PIECE // 16)
        def _(g):
            dvec = dv[pl.ds(g * 16, 16)]
            for l in range(16):
                r = g * 16 + l
                dsc = dvec[l]
                a0 = aggv[r, pl.ds(0, 16)]
                y0 = yv[r, pl.ds(0, 16)]
                aggv[r, pl.ds(0, 16)] = jnp.maximum(dsc * (a0 + y0) + b1a, 0.0)
                a1 = aggv[r, pl.ds(16, 16)]
                y1 = yv[r, pl.ds(16, 16)]
                aggv[r, pl.ds(16, 16)] = jnp.maximum(dsc * (a1 + y1) + b1b, 0.0)

        pltpu.sync_copy(aggv, sums_sh.at[bv], add=True)
        pltpu.sync_copy(onesv, cnt_sh.at[bv], add=True)

    plsc.subcore_barrier()

    @pl.when(t == 0)
    def _():
        pltpu.sync_copy(sums_sh, rows_b.at[0, pl.ds(0, G)])
        pltpu.sync_copy(rows_b.at[0, pl.ds(0, G)], sums_out.at[c])

    @pl.when(jnp.logical_and(t == 1, c == 0))
    def _():
        pltpu.sync_copy(cnt_sh, zv)
        pltpu.sync_copy(zv, cnt_out)


@functools.cache
def _gnn_call():
    return pl.kernel(
        _gnn_body,
        out_type=(
            jax.ShapeDtypeStruct((NC, G, CH), _f32),
            jax.ShapeDtypeStruct((G,), _f32),
        ),
        mesh=_mesh(),
        compiler_params=_SC_PARAMS,
        scratch_types=[
            pltpu.VMEM_SHARED((N, CH), _f32),
            pltpu.VMEM_SHARED((G, CH), _f32),
            pltpu.VMEM_SHARED((G,), _f32),
            pltpu.VMEM((FB,), _i32),
            pltpu.VMEM((FB,), _i32),
            pltpu.VMEM((FB,), _i32),
            pltpu.VMEM((FB,), _i32),
            pltpu.VMEM((FB,), _i32),
            pltpu.VMEM((FB,), _i32),
            pltpu.VMEM((FB,), _i32),
            pltpu.VMEM((FB,), _i32),
            pltpu.VMEM((2, FB, CH), _f32),
            pltpu.VMEM((PIECE, CH), _f32),
            pltpu.VMEM((PIECE, CH), _f32),
            pltpu.VMEM((PIECE,), _f32),
            pltpu.VMEM((PIECE,), _i32),
            pltpu.VMEM((PIECE,), _f32),
            pltpu.VMEM((G,), _f32),
            pltpu.VMEM((CH,), _f32),
            pltpu.SemaphoreType.DMA,
            pltpu.SemaphoreType.DMA,
            pltpu.SemaphoreType.DMA,
        ],
    )


def _fc_body(sums_ref, cnt_ref, w2_ref, b2_ref, out_ref):
    s = sums_ref[...]
    w2 = w2_ref[...]
    num = jnp.dot(s[0], w2[0:CH, :], preferred_element_type=_f32)
    num = num + jnp.dot(s[1], w2[CH:D_HID, :], preferred_element_type=_f32)
    cnt = jnp.maximum(cnt_ref[...], 1.0)
    out_ref[...] = num / cnt + b2_ref[...]


_fc_call = pl.pallas_call(
    _fc_body,
    out_shape=jax.ShapeDtypeStruct((G, 1), _f32),
)


def kernel(x, edge_index, batch, W1, b1, W2, b2):
    pdeg = _deg_call()(edge_index[1])
    y3, dinv3 = _mm_call(x, W1, pdeg.reshape(NC, MB, 1, BM))
    yflat = y3.reshape(NC * N, CH)
    dinv = dinv3.reshape(N)
    sums, counts = _gnn_call()(edge_index, yflat, dinv, batch, b1)
    return _fc_call(sums, counts.reshape(G, 1), W2, b2.reshape(1, 1))


# speedup vs baseline: 41.0588x; 1.0609x over previous
import functools

import jax
import jax.numpy as jnp
from jax import lax
from jax.experimental import pallas as pl
from jax.experimental.pallas import tpu as pltpu
from jax.experimental.pallas import tpu_sc as plsc

N = 50000
E = 800000
D_IN = 79
D_HID = 64
G = 256
NC = 2
NS = 16
CH = D_HID // NC
CB = 128
NCHUNK = E // CB
FB = 320
NFAT = E // FB
DPIECE = 400
PIECE = 80
BM = 2000
MB = N // BM

_f32 = jnp.float32
_i32 = jnp.int32


@functools.cache
def _mesh():
    return plsc.VectorSubcoreMesh(
        core_axis_name="c", subcore_axis_name="s", num_cores=NC, num_subcores=NS
    )


_SC_PARAMS = pltpu.CompilerParams(use_tc_tiling_on_sc=False)


def _fill_1d(ref, n, value):
    v = jnp.full((16,), value, ref.dtype)

    @pl.loop(0, n // 16)
    def _(i):
        ref[pl.ds(i * 16, 16)] = v


def _fill_2d(ref, nrows, value):
    v = jnp.full((16,), value, ref.dtype)

    @pl.loop(0, nrows)
    def _(r):
        for h in range(CH // 16):
            ref[r, pl.ds(16 * h, 16)] = v


def _split(total, t, nbig):
    base = total // NS
    start = base * t + jnp.minimum(t, nbig)
    cnt = base + (t < nbig).astype(_i32)
    return start, cnt


def _splitw(total, w, n):
    base = total // n
    nbig = total % n
    start = base * w + jnp.minimum(w, nbig)
    cnt = base + (w < nbig).astype(_i32)
    return start, cnt


def _deg_body(dst1d, pdeg, deg_sh, idst0, idst1, ones_v, piece_v, sem):
    c = lax.axis_index("c")
    t = lax.axis_index("s")
    _fill_1d(piece_v, DPIECE, 0.0)
    _fill_1d(ones_v, FB, 1.0)
    p0, pcnt = _split(N // DPIECE, t, (N // DPIECE) % NS)

    @pl.loop(p0, p0 + pcnt)
    def _(p):
        pltpu.sync_copy(piece_v, deg_sh.at[pl.ds(p * DPIECE, DPIECE)])

    plsc.subcore_barrier()

    w = c * NS + t
    f0, cnt = _splitw(NFAT, w, NC * NS)
    npairs = cnt // 2

    def _idx(j):
        return dst1d.at[pl.ds(j * FB, FB)]

    pltpu.async_copy(_idx(f0), idst0, sem)

    @pl.loop(0, npairs)
    def _(i):
        j = f0 + 2 * i
        pltpu.make_async_copy(_idx(j), idst0, sem).wait()
        pltpu.async_copy(_idx(j + 1), idst1, sem)
        pltpu.sync_copy(ones_v, deg_sh.at[idst0], add=True)
        pltpu.make_async_copy(_idx(j + 1), idst1, sem).wait()

        @pl.when(2 * i + 2 < cnt)
        def _():
            pltpu.async_copy(_idx(j + 2), idst0, sem)

        pltpu.sync_copy(ones_v, deg_sh.at[idst1], add=True)

    @pl.when(cnt % 2 == 1)
    def _():
        jl = f0 + cnt - 1
        pltpu.make_async_copy(_idx(jl), idst0, sem).wait()
        pltpu.sync_copy(ones_v, deg_sh.at[idst0], add=True)

    plsc.subcore_barrier()

    @pl.loop(p0, p0 + pcnt)
    def _(p):
        pltpu.sync_copy(deg_sh.at[pl.ds(p * DPIECE, DPIECE)], piece_v)
        pltpu.sync_copy(piece_v, pdeg.at[pl.ds(c * N + p * DPIECE, DPIECE)])


@functools.cache
def _deg_call():
    return pl.kernel(
        _deg_body,
        out_type=jax.ShapeDtypeStruct((NC * N,), _f32),
        mesh=_mesh(),
        compiler_params=_SC_PARAMS,
        scratch_types=[
            pltpu.VMEM_SHARED((N,), _f32),
            pltpu.VMEM((FB,), _i32),
            pltpu.VMEM((FB,), _i32),
            pltpu.VMEM((FB,), _f32),
            pltpu.VMEM((DPIECE,), _f32),
            pltpu.SemaphoreType.DMA,
        ],
    )


def _mm_body(x_ref, w1_ref, pdeg_ref, y_ref, dinv_ref):
    pd = pdeg_ref[...]
    deg = pd[0, 0, 0, :] + pd[1, 0, 0, :] + 1.0
    dinv = lax.rsqrt(deg)
    dinv_ref[...] = dinv.reshape(1, 1, BM)
    xw = jnp.dot(x_ref[...], w1_ref[0], preferred_element_type=_f32)
    y_ref[...] = xw * dinv[:, None]


_mm_call = pl.pallas_call(
    _mm_body,
    grid=(MB, NC),
    in_specs=[
        pl.BlockSpec((BM, D_IN), lambda m, c: (m, 0)),
        pl.BlockSpec((1, D_IN, CH), lambda m, c: (c, 0, 0)),
        pl.BlockSpec((NC, 1, 1, BM), lambda m, c: (0, m, 0, 0)),
    ],
    out_specs=[
        pl.BlockSpec((BM, CH), lambda m, c: (c * MB + m, 0)),
        pl.BlockSpec((1, 1, BM), lambda m, c: (m, 0, 0)),
    ],
    out_shape=[
        jax.ShapeDtypeStruct((NC * N, CH), _f32),
        jax.ShapeDtypeStruct((MB, 1, BM), _f32),
    ],
)


def _gnn_body(
    edges4, yflat, dinv, batch, b1,
    sums_out, cnt_out,
    agg_sh, sums_sh, cnt_sh,
    isrc0, isrc1, isrc2, isrc3, idst0, idst1, idst2, idst3, rows_b,
    aggv, yv, dv, bv, onesv, zv, b1v, sem, sem_s, sem_i,
):
    c = lax.axis_index("c")
    t = lax.axis_index("s")
    coff = c * N

    _fill_2d(aggv, PIECE, 0.0)
    _fill_1d(zv, G, 0.0)
    _fill_1d(onesv, PIECE, 1.0)
    p0, pcnt = _split(N // PIECE, t, (N // PIECE) % NS)

    @pl.loop(p0, p0 + pcnt)
    def _(p):
        pltpu.sync_copy(aggv, agg_sh.at[pl.ds(p * PIECE, PIECE)])

    pltpu.sync_copy(
        aggv.at[pl.ds(0, G // NS)], sums_sh.at[pl.ds(t * (G // NS), G // NS)]
    )

    @pl.when(t == 0)
    def _():
        pltpu.sync_copy(zv, cnt_sh)

    pltpu.sync_copy(b1.at[pl.ds(c * CH, CH)], b1v)
    plsc.subcore_barrier()

    nfq = NFAT // 4
    q0, qcnt = _split(nfq, t, nfq % NS)
    f0 = 4 * q0
    offv = jnp.full((16,), coff, _i32)
    isrcs = (isrc0, isrc1, isrc2, isrc3)
    idsts = (idst0, idst1, idst2, idst3)

    def _fire_idx(f, r):
        pltpu.async_copy(edges4.at[0, pl.ds(f * FB, FB)], isrcs[r], sem_i)
        pltpu.async_copy(edges4.at[1, pl.ds(f * FB, FB)], idsts[r], sem_i)

    def _wait_idx_adjust(f, r):
        si = isrcs[r]
        pltpu.make_async_copy(edges4.at[0, pl.ds(f * FB, FB)], si, sem_i).wait()
        pltpu.make_async_copy(
            edges4.at[1, pl.ds(f * FB, FB)], idsts[r], sem_i
        ).wait()
        for k in range(FB // 16):
            si[pl.ds(k * 16, 16)] = si[pl.ds(k * 16, 16)] + offv

    def _wait_g(r, p):
        pltpu.make_async_copy(yflat.at[isrcs[r]], rows_b.at[p], sem).wait()

    def _fire_g(r, p):
        pltpu.async_copy(yflat.at[isrcs[r]], rows_b.at[p], sem)

    def _fire_s(r, p):
        pltpu.make_async_copy(rows_b.at[p], agg_sh.at[idsts[r]], sem_s).start(
            add=True
        )

    def _wait_s(r, p):
        pltpu.make_async_copy(rows_b.at[p], agg_sh.at[idsts[r]], sem_s).wait()

    for k in range(3):
        _fire_idx(f0 + k, k)
    _wait_idx_adjust(f0, 0)
    _fire_g(0, 0)

    @pl.loop(0, qcnt)
    def _(i):
        for k in range(4):
            p = k % 2
            _wait_g(k, p)
            _fire_s(k, p)
            if k == 0:
                @pl.when(i > 0)
                def _():
                    _wait_s(3, 1)
            else:
                _wait_s(k - 1, 1 - p)
            if k == 3:
                @pl.when(i < qcnt - 1)
                def _():
                    _wait_idx_adjust(f0 + 4 * i + k + 1, 0)
                    _fire_g(0, 1 - p)
            else:
                _wait_idx_adjust(f0 + 4 * i + k + 1, (k + 1) % 4)
                _fire_g((k + 1) % 4, 1 - p)
            if k == 0:
                _fire_idx(f0 + 4 * i + k + 3, 3)
            else:
                @pl.when(i < qcnt - 1)
                def _():
                    _fire_idx(f0 + 4 * i + k + 3, (k + 3) % 4)

    _wait_s(3, 1)

    plsc.subcore_barrier()

    b1a = b1v[pl.ds(0, 16)]
    b1b = b1v[pl.ds(16, 16)]

    @pl.loop(p0, p0 + pcnt)
    def _(p):
        row0 = p * PIECE
        pltpu.sync_copy(agg_sh.at[pl.ds(row0, PIECE)], aggv)
        pltpu.sync_copy(yflat.at[pl.ds(coff + row0, PIECE)], yv)
        pltpu.sync_copy(dinv.at[pl.ds(row0, PIECE)], dv)
        pltpu.sync_copy(batch.at[pl.ds(row0, PIECE)], bv)

        @pl.loop(0, PIECE // 16)
        def _(g):
            dvec = dv[pl.ds(g * 16, 16)]
            for l in range(16):
                r = g * 16 + l
                dsc = dvec[l]
                a0 = aggv[r, pl.ds(0, 16)]
                y0 = yv[r, pl.ds(0, 16)]
                aggv[r, pl.ds(0, 16)] = jnp.maximum(dsc * (a0 + y0) + b1a, 0.0)
                a1 = aggv[r, pl.ds(16, 16)]
                y1 = yv[r, pl.ds(16, 16)]
                aggv[r, pl.ds(16, 16)] = jnp.maximum(dsc * (a1 + y1) + b1b, 0.0)

        pltpu.sync_copy(aggv, sums_sh.at[bv], add=True)
        pltpu.sync_copy(onesv, cnt_sh.at[bv], add=True)

    plsc.subcore_barrier()

    @pl.when(t == 0)
    def _():
        pltpu.sync_copy(sums_sh, rows_b.at[0, pl.ds(0, G)])
        pltpu.sync_copy(rows_b.at[0, pl.ds(0, G)], sums_out.at[c])

    @pl.when(jnp.logical_and(t == 1, c == 0))
    def _():
        pltpu.sync_copy(cnt_sh, zv)
        pltpu.sync_copy(zv, cnt_out)


@functools.cache
def _gnn_call():
    return pl.kernel(
        _gnn_body,
        out_type=(
            jax.ShapeDtypeStruct((NC, G, CH), _f32),
            jax.ShapeDtypeStruct((G,), _f32),
        ),
        mesh=_mesh(),
        compiler_params=_SC_PARAMS,
        scratch_types=[
            pltpu.VMEM_SHARED((N, CH), _f32),
            pltpu.VMEM_SHARED((G, CH), _f32),
            pltpu.VMEM_SHARED((G,), _f32),
            pltpu.VMEM((FB,), _i32),
            pltpu.VMEM((FB,), _i32),
            pltpu.VMEM((FB,), _i32),
            pltpu.VMEM((FB,), _i32),
            pltpu.VMEM((FB,), _i32),
            pltpu.VMEM((FB,), _i32),
            pltpu.VMEM((FB,), _i32),
            pltpu.VMEM((FB,), _i32),
            pltpu.VMEM((2, FB, CH), _f32),
            pltpu.VMEM((PIECE, CH), _f32),
            pltpu.VMEM((PIECE, CH), _f32),
            pltpu.VMEM((PIECE,), _f32),
            pltpu.VMEM((PIECE,), _i32),
            pltpu.VMEM((PIECE,), _f32),
            pltpu.VMEM((G,), _f32),
            pltpu.VMEM((CH,), _f32),
            pltpu.SemaphoreType.DMA,
            pltpu.SemaphoreType.DMA,
            pltpu.SemaphoreType.DMA,
        ],
    )


def _fc_body(sums_ref, cnt_ref, w2_ref, b2_ref, out_ref):
    s = sums_ref[...]
    w2 = w2_ref[...]
    num = jnp.dot(s[0], w2[0:CH, :], preferred_element_type=_f32)
    num = num + jnp.dot(s[1], w2[CH:D_HID, :], preferred_element_type=_f32)
    cnt = jnp.maximum(cnt_ref[...], 1.0)
    out_ref[...] = num / cnt + b2_ref[...]


_fc_call = pl.pallas_call(
    _fc_body,
    out_shape=jax.ShapeDtypeStruct((G, 1), _f32),
)


def kernel(x, edge_index, batch, W1, b1, W2, b2):
    pdeg = _deg_call()(edge_index[1])
    w1s = W1.reshape(D_IN, NC, CH).transpose(1, 0, 2)
    yflat, dinv3 = _mm_call(x, w1s, pdeg.reshape(NC, MB, 1, BM))
    dinv = dinv3.reshape(N)
    sums, counts = _gnn_call()(edge_index, yflat, dinv, batch, b1)
    return _fc_call(sums, counts.reshape(G, 1), W2, b2.reshape(1, 1))
